# baseline (speedup 1.0000x reference)
import jax
import jax.numpy as jnp
from jax.experimental import pallas as pl
from jax.experimental.pallas import tpu as pltpu


def _vq_body(zt_ref, ct_ref, cbbf_ref, zq_ref, idx_ref, prev_ref, cn_ref):
    i = pl.program_id(0)
    nsteps = pl.num_programs(0)
    k_codes = ct_ref.shape[1]
    dn = (((1,), (0,)), ((), ()))

    @pl.when(i == 0)
    def _init_cnorm():
        ct = ct_ref[...]
        cn_ref[...] = jnp.sum(ct * ct, axis=0, keepdims=True)

    @pl.when(i > 0)
    def _gather_prev():
        idxp = prev_ref[...]
        iig = jax.lax.broadcasted_iota(jnp.int32, (idxp.shape[0], k_codes), 1)
        oh = (iig == idxp).astype(jnp.bfloat16)
        zq = jax.lax.dot_general(oh, cbbf_ref[...], dn,
                                 preferred_element_type=jnp.float32)
        zq_ref[0] = zq
        idx_ref[0] = idxp.reshape(1, idxp.shape[0])

    @pl.when(i < nsteps - 1)
    def _score_cur():
        zf = zt_ref[0]
        ct = ct_ref[...]
        s = jax.lax.dot_general(zf, ct, dn, preferred_element_type=jnp.float32)
        znorm = jnp.sum(zf * zf, axis=1, keepdims=True)
        d = (znorm - 2.0 * s) + cn_ref[...]

        minv = jnp.min(d, axis=1, keepdims=True)
        ii = jax.lax.broadcasted_iota(jnp.int32, d.shape, 1)
        idx = jnp.min(jnp.where(d == minv, ii, k_codes), axis=1, keepdims=True)
        prev_ref[...] = idx


def kernel(z, codebook):
    b, c, h, w = z.shape
    hw = h * w
    k = codebook.shape[0]
    zt = z.transpose(0, 2, 3, 1).reshape(b, hw, c)
    ct = codebook.T
    cb_bf = codebook.astype(jnp.bfloat16)

    zq3, idx3 = pl.pallas_call(
        _vq_body,
        grid=(b + 1,),
        in_specs=[
            pl.BlockSpec((1, hw, c), lambda i: (jnp.minimum(i, b - 1), 0, 0)),
            pl.BlockSpec((c, k), lambda i: (0, 0)),
            pl.BlockSpec((k, c), lambda i: (0, 0)),
        ],
        out_specs=[
            pl.BlockSpec((1, hw, c), lambda i: (jnp.maximum(i - 1, 0), 0, 0)),
            pl.BlockSpec((1, 1, hw), lambda i: (jnp.maximum(i - 1, 0), 0, 0)),
        ],
        out_shape=[
            jax.ShapeDtypeStruct((b, hw, c), jnp.float32),
            jax.ShapeDtypeStruct((b, 1, hw), jnp.int32),
        ],
        scratch_shapes=[
            pltpu.VMEM((hw, 1), jnp.int32),
            pltpu.VMEM((1, k), jnp.float32),
        ],
    )(zt, ct, cb_bf)
    zq = zq3.reshape(b, h, w, c).transpose(0, 3, 1, 2)
    return zq, idx3.reshape(b, hw)


# speedup vs baseline: 1.0400x; 1.0400x over previous
import jax
import jax.numpy as jnp
from jax.experimental import pallas as pl


def _vq_body(zt_ref, ct_ref, cbbf_ref, zq_ref, idx_ref):
    zf = zt_ref[0]
    ct = ct_ref[...]
    k_codes = ct.shape[1]
    dn = (((1,), (0,)), ((), ()))

    s = jax.lax.dot_general(zf, ct, dn, preferred_element_type=jnp.float32)
    znorm = jnp.sum(zf * zf, axis=1, keepdims=True)
    cnorm = jnp.sum(ct * ct, axis=0, keepdims=True)
    d = (znorm - 2.0 * s) + cnorm

    idx = jnp.argmin(d, axis=1).astype(jnp.int32)
    idxc = idx[:, None]

    ii = jax.lax.broadcasted_iota(jnp.int32, d.shape, 1)
    oh = (ii == idxc).astype(jnp.bfloat16)
    zq = jax.lax.dot_general(oh, cbbf_ref[...], dn,
                             preferred_element_type=jnp.float32)
    zq_ref[0] = zq
    idx_ref[0] = idx[None, :]


def kernel(z, codebook):
    b, c, h, w = z.shape
    hw = h * w
    k = codebook.shape[0]
    zt = z.transpose(0, 2, 3, 1).reshape(b, hw, c)
    ct = codebook.T
    cb_bf = codebook.astype(jnp.bfloat16)

    zq3, idx3 = pl.pallas_call(
        _vq_body,
        grid=(b,),
        in_specs=[
            pl.BlockSpec((1, hw, c), lambda i: (i, 0, 0)),
            pl.BlockSpec((c, k), lambda i: (0, 0)),
            pl.BlockSpec((k, c), lambda i: (0, 0)),
        ],
        out_specs=[
            pl.BlockSpec((1, hw, c), lambda i: (i, 0, 0)),
            pl.BlockSpec((1, 1, hw), lambda i: (i, 0, 0)),
        ],
        out_shape=[
            jax.ShapeDtypeStruct((b, hw, c), jnp.float32),
            jax.ShapeDtypeStruct((b, 1, hw), jnp.int32),
        ],
    )(zt, ct, cb_bf)
    zq = zq3.reshape(b, h, w, c).transpose(0, 3, 1, 2)
    return zq, idx3.reshape(b, hw)
